# baseline (speedup 1.0000x reference)
import functools

import jax
import jax.numpy as jnp
from jax import lax
from jax.experimental import pallas as pl
from jax.experimental.pallas import tpu as pltpu
from jax.experimental.pallas import tpu_sc as plsc

LANE = 128
H = 32


@functools.lru_cache(maxsize=None)
def _make_transpose(vt):
    info = plsc.get_sparse_core_info()
    nc, ns = info.num_cores, info.num_subcores
    nw = nc * ns
    per_w = (vt + nw - 1) // nw
    mesh = plsc.VectorSubcoreMesh(core_axis_name="c", subcore_axis_name="s")

    @functools.partial(
        pl.kernel,
        mesh=mesh,
        compiler_params=pltpu.CompilerParams(use_tc_tiling_on_sc=True, needs_layout_passes=False),
        out_type=jax.ShapeDtypeStruct((vt * 32, LANE), jnp.float32),
        scratch_types=[
            pltpu.VMEM((4, 8, LANE), jnp.float32),
            pltpu.VMEM((32, LANE), jnp.float32),
        ],
    )
    def kern(tt_hbm, out_hbm, in_v, out_v):
        wid = lax.axis_index("s") * nc + lax.axis_index("c")
        iota = lax.iota(jnp.int32, 16)
        fa_vecs = [(16 * (k % 2) + iota) >> 3 for k in range(2)]
        fr_vec = iota & 7

        def block(i, carry):
            tc = wid * per_w + i

            @pl.when(tc < vt)
            def _():
                for fa in range(4):
                    pltpu.sync_copy(
                        tt_hbm.at[pl.ds(fa * 8, 8), pl.ds(tc * LANE, LANE)],
                        in_v.at[fa],
                    )
                for q in range(32):
                    for k in range(8):
                        col = jnp.full((16,), 4 * q + k // 2, jnp.int32)
                        vec = plsc.load_gather(
                            in_v, [fa_vecs[k % 2], fr_vec, col]
                        )
                        out_v[q, pl.ds(16 * k, 16)] = vec
                pltpu.sync_copy(out_v, out_hbm.at[pl.ds(tc * 32, 32)])

            return carry

        lax.fori_loop(0, per_w, block, 0)

    return kern


@functools.lru_cache(maxsize=None)
def _make_gather(na, nb, vp):
    info = plsc.get_sparse_core_info()
    nc, ns = info.num_cores, info.num_subcores
    nw = nc * ns
    n_tiles = na * nb
    per_w = n_tiles // nw
    assert n_tiles % nw == 0
    mesh = plsc.VectorSubcoreMesh(core_axis_name="c", subcore_axis_name="s")

    @functools.partial(
        pl.kernel,
        mesh=mesh,
        compiler_params=pltpu.CompilerParams(use_tc_tiling_on_sc=False, needs_layout_passes=False),
        out_type=jax.ShapeDtypeStruct((8 * na, 4, nb, 8, LANE), jnp.float32),
        scratch_types=[
            pltpu.VMEM((8, LANE), jnp.int32),
            pltpu.VMEM((LANE, H), jnp.float32),
            pltpu.VMEM((4, 8, LANE), jnp.float32),
            pltpu.SemaphoreType.DMA,
        ],
    )
    def kern(x4_hbm, tab_hbm, out_hbm, idx_v, rows_v, tr_v, gsem):
        wid = lax.axis_index("s") * nc + lax.axis_index("c")
        iota = lax.iota(jnp.int32, 16)
        row_vecs = [16 * k + iota for k in range(8)]

        def tile(t, carry):
            tt = wid * per_w + t
            a = tt // nb
            b = tt % nb
            pltpu.sync_copy(x4_hbm.at[a, b], idx_v)

            def group(r, carry2):
                j = 8 * a + r
                pltpu.async_copy(
                    tab_hbm.at[idx_v.at[r]], rows_v, gsem
                ).wait()
                for f in range(H):
                    col = jnp.full((16,), f, jnp.int32)
                    for k in range(8):
                        vec = plsc.load_gather(rows_v, [row_vecs[k], col])
                        tr_v[f // 8, f % 8, pl.ds(16 * k, 16)] = vec
                for fa in range(4):
                    pltpu.sync_copy(tr_v.at[fa], out_hbm.at[j, fa, b])
                return carry2

            lax.fori_loop(0, 8, group, 0)
            return carry

        lax.fori_loop(0, per_w, tile, 0)

    return kern


def kernel(x, table):
    n_tok, n_pos = x.shape
    v, d = table.shape
    assert d == H and n_tok % LANE == 0 and n_pos % 8 == 0
    na, nb = n_pos // 8, n_tok // LANE
    vp = (v + LANE - 1) // LANE * LANE
    vt = vp // LANE

    x4 = x.T.reshape(na, 8, nb, LANE).transpose(0, 2, 1, 3).astype(jnp.int32)
    t128 = _make_transpose(vt)(table.T)
    t_lin = t128.reshape(vp, d)

    o5 = _make_gather(na, nb, vp)(x4, t_lin)
    out = (
        o5.transpose(0, 1, 3, 2, 4)
        .reshape(n_pos, d, n_tok)
        .transpose(2, 0, 1)
    )
    return out


# speedup vs baseline: 1.4031x; 1.4031x over previous
import functools

import jax
import jax.numpy as jnp
from jax import lax
from jax.experimental import pallas as pl
from jax.experimental.pallas import tpu as pltpu
from jax.experimental.pallas import tpu_sc as plsc

LANE = 128
H = 32


@functools.lru_cache(maxsize=None)
def _make_transpose(vt):
    info = plsc.get_sparse_core_info()
    nc, ns = info.num_cores, info.num_subcores
    nw = nc * ns
    per_w = (vt + nw - 1) // nw
    mesh = plsc.VectorSubcoreMesh(core_axis_name="c", subcore_axis_name="s")

    @functools.partial(
        pl.kernel,
        mesh=mesh,
        compiler_params=pltpu.CompilerParams(use_tc_tiling_on_sc=True, needs_layout_passes=False),
        out_type=jax.ShapeDtypeStruct((vt * 32, LANE), jnp.float32),
        scratch_types=[
            pltpu.VMEM((4, 8, LANE), jnp.float32),
            pltpu.VMEM((32, LANE), jnp.float32),
        ],
    )
    def kern(tt_hbm, out_hbm, in_v, out_v):
        wid = lax.axis_index("s") * nc + lax.axis_index("c")
        iota = lax.iota(jnp.int32, 16)
        fa_vecs = [(16 * h + iota) >> 3 for h in range(2)]
        fr_vec = iota & 7

        def block(i, carry):
            tc = wid * per_w + i

            @pl.when(tc < vt)
            def _():
                for fa in range(4):
                    pltpu.sync_copy(
                        tt_hbm.at[pl.ds(fa * 8, 8), pl.ds(tc * LANE, LANE)],
                        in_v.at[fa],
                    )

                @plsc.parallel_loop(0, LANE, 1, unroll=4)
                def _(v_loc):
                    col = jnp.full((16,), 0, jnp.int32) + v_loc
                    q = v_loc >> 2
                    base = (v_loc & 3) * 32
                    for h in range(2):
                        vec = plsc.load_gather(in_v, [fa_vecs[h], fr_vec, col])
                        out_v[q, pl.ds(base + 16 * h, 16)] = vec

                pltpu.sync_copy(out_v, out_hbm.at[pl.ds(tc * 32, 32)])

            return carry

        lax.fori_loop(0, per_w, block, 0)

    return kern


@functools.lru_cache(maxsize=None)
def _make_gather(na, nb, vp):
    info = plsc.get_sparse_core_info()
    nc, ns = info.num_cores, info.num_subcores
    nw = nc * ns
    n_tiles = na * nb
    per_w = n_tiles // nw
    assert n_tiles % nw == 0
    mesh = plsc.VectorSubcoreMesh(core_axis_name="c", subcore_axis_name="s")

    @functools.partial(
        pl.kernel,
        mesh=mesh,
        compiler_params=pltpu.CompilerParams(use_tc_tiling_on_sc=False, needs_layout_passes=False),
        out_type=jax.ShapeDtypeStruct((8 * na, 4, nb, 8, LANE), jnp.float32),
        scratch_types=[
            pltpu.VMEM((8, LANE), jnp.int32),
            pltpu.VMEM((LANE, H), jnp.float32),
            pltpu.VMEM((H, LANE), jnp.float32),
            pltpu.SemaphoreType.DMA,
        ],
    )
    def kern(x4_hbm, tab_hbm, out_hbm, idx_v, rows_v, tr_v, gsem):
        wid = lax.axis_index("s") * nc + lax.axis_index("c")
        iota = lax.iota(jnp.int32, 16)
        f_vecs = [16 * h + iota for h in range(2)]

        def tile(t, carry):
            tt = wid * per_w + t
            a = tt // nb
            b = tt % nb
            pltpu.sync_copy(x4_hbm.at[a, b], idx_v)

            def group(r, carry2):
                j = 8 * a + r
                pltpu.async_copy(
                    tab_hbm.at[idx_v.at[r]], rows_v, gsem
                ).wait()

                @plsc.parallel_loop(0, LANE, 1, unroll=4)
                def _(c):
                    col = jnp.full((16,), 0, jnp.int32) + c
                    for h in range(2):
                        vec = rows_v[c, pl.ds(16 * h, 16)]
                        plsc.store_scatter(tr_v, [f_vecs[h], col], vec)

                for fa in range(4):
                    pltpu.sync_copy(
                        tr_v.at[pl.ds(fa * 8, 8)], out_hbm.at[j, fa, b]
                    )
                return carry2

            lax.fori_loop(0, 8, group, 0)
            return carry

        lax.fori_loop(0, per_w, tile, 0)

    return kern


def kernel(x, table):
    n_tok, n_pos = x.shape
    v, d = table.shape
    assert d == H and n_tok % LANE == 0 and n_pos % 8 == 0
    na, nb = n_pos // 8, n_tok // LANE
    vp = (v + LANE - 1) // LANE * LANE
    vt = vp // LANE

    x4 = x.T.reshape(na, 8, nb, LANE).transpose(0, 2, 1, 3).astype(jnp.int32)
    t128 = _make_transpose(vt)(table.T)
    t_lin = t128.reshape(vp, d)

    o5 = _make_gather(na, nb, vp)(x4, t_lin)
    out = (
        o5.transpose(0, 1, 3, 2, 4)
        .reshape(n_pos, d, n_tok)
        .transpose(2, 0, 1)
    )
    return out


# speedup vs baseline: 2.1735x; 1.5491x over previous
import functools

import jax
import jax.numpy as jnp
from jax import lax
from jax.experimental import pallas as pl
from jax.experimental.pallas import tpu as pltpu
from jax.experimental.pallas import tpu_sc as plsc

LANE = 128
H = 32


@functools.lru_cache(maxsize=None)
def _make_gather(na, nb, v):
    info = plsc.get_sparse_core_info()
    nc, ns = info.num_cores, info.num_subcores
    nw = nc * ns
    n_tiles = na * nb
    per_w = n_tiles // nw
    assert n_tiles % nw == 0
    mesh = plsc.VectorSubcoreMesh(core_axis_name="c", subcore_axis_name="s")

    @functools.partial(
        pl.kernel,
        mesh=mesh,
        compiler_params=pltpu.CompilerParams(use_tc_tiling_on_sc=False),
        out_type=jax.ShapeDtypeStruct((8 * na, LANE * nb, H), jnp.float32),
        scratch_types=[
            pltpu.VMEM((2, 8, LANE), jnp.int32),
            pltpu.VMEM((2, 8, LANE, H), jnp.float32),
            pltpu.SemaphoreType.DMA((2,)),
            pltpu.SemaphoreType.DMA((2,)),
            pltpu.SemaphoreType.DMA((2,)),
        ],
    )
    def kern(x4_hbm, tab_hbm, out_hbm, idx_v, rows_v, isem, gsem, osem):
        wid = lax.axis_index("s") * nc + lax.axis_index("c")

        def idx_src(t):
            tt = wid * per_w + t
            return x4_hbm.at[tt // nb, tt % nb]

        def fire_gathers(t, sl):
            for r in range(8):
                pltpu.async_copy(
                    tab_hbm.at[idx_v.at[sl, r]],
                    rows_v.at[sl, r],
                    gsem.at[sl],
                )

        def drain_and_store(t, sl):
            tt = wid * per_w + t
            a = tt // nb
            b = tt % nb
            for r in range(8):
                pltpu.make_async_copy(
                    tab_hbm.at[idx_v.at[sl, r]], rows_v.at[sl, r], gsem.at[sl]
                ).wait()
            for r in range(8):
                pltpu.async_copy(
                    rows_v.at[sl, r],
                    out_hbm.at[8 * a + r, pl.ds(b * LANE, LANE)],
                    osem.at[sl],
                )

        def wait_stores(t, sl):
            tt = wid * per_w + t
            a = tt // nb
            b = tt % nb
            for r in range(8):
                pltpu.make_async_copy(
                    rows_v.at[sl, r],
                    out_hbm.at[8 * a + r, pl.ds(b * LANE, LANE)],
                    osem.at[sl],
                ).wait()

        pltpu.async_copy(idx_src(0), idx_v.at[0], isem.at[0])

        def step(t, carry):
            sl = t % 2
            nsl = (t + 1) % 2

            @pl.when(t + 1 < per_w)
            def _():
                pltpu.async_copy(idx_src(t + 1), idx_v.at[nsl], isem.at[nsl])

            pltpu.make_async_copy(idx_src(t), idx_v.at[sl], isem.at[sl]).wait()

            @pl.when(t >= 2)
            def _():
                wait_stores(t - 2, sl)

            fire_gathers(t, sl)
            drain_and_store(t, sl)
            return carry

        lax.fori_loop(0, per_w, step, 0)
        for t in (per_w - 2, per_w - 1):
            wait_stores(t, t % 2)

    return kern


def kernel(x, table):
    n_tok, n_pos = x.shape
    v, d = table.shape
    assert d == H and n_tok % LANE == 0 and n_pos % 8 == 0
    na, nb = n_pos // 8, n_tok // LANE

    x4 = x.T.reshape(na, 8, nb, LANE).transpose(0, 2, 1, 3).astype(jnp.int32)
    oj = _make_gather(na, nb, v)(x4, table)
    return oj.transpose(1, 0, 2)
